# baseline (speedup 1.0000x reference)
import functools

import jax
import jax.numpy as jnp
from jax import lax
from jax.experimental import pallas as pl
from jax.experimental.pallas import tpu as pltpu
from jax.experimental.pallas import tpu_sc as plsc


def kernel(frames):
    B, C, T, H, W = frames.shape
    S = T // 4
    BC = B * C
    ROWS = BC * S
    NW = 32
    RPW = ROWS // NW
    HH = H // 2
    TOT = RPW * 2
    NBUF = 4
    AHEAD = 2

    src = frames.reshape(BC, T, H, W)
    mesh = plsc.VectorSubcoreMesh(core_axis_name="c", subcore_axis_name="s")

    @functools.partial(
        pl.kernel,
        out_type=jax.ShapeDtypeStruct((BC, S, H, W), frames.dtype),
        mesh=mesh,
        scratch_types=[
            pltpu.VMEM((NBUF, HH, W), frames.dtype),
            pltpu.SemaphoreType.DMA((NBUF,)),
            pltpu.SemaphoreType.DMA((NBUF,)),
        ],
    )
    def pack_slow(src_hbm, out_hbm, buf, sin, sout):
        wid = lax.axis_index("s") * 2 + lax.axis_index("c")
        base = wid * RPW

        def gather(j):
            r = base + (j >> 1)
            tp = r % S
            h0 = (j & 1) * HH
            return pltpu.make_async_copy(
                src_hbm.at[r // S, (tp * 567) >> 7, pl.ds(h0, HH)],
                buf.at[j % NBUF], sin.at[j % NBUF])

        def scatter(j):
            r = base + (j >> 1)
            h0 = (j & 1) * HH
            return pltpu.make_async_copy(
                buf.at[j % NBUF],
                out_hbm.at[r // S, r % S, pl.ds(h0, HH)],
                sout.at[j % NBUF])

        waited = set()
        for j in range(AHEAD):
            gather(j).start()
        for j in range(TOT):
            gather(j).wait()
            scatter(j).start()
            k = j + AHEAD
            if k < TOT:
                p = k - NBUF
                if p >= 0:
                    scatter(p).wait()
                    waited.add(p)
                gather(k).start()
        for j in range(TOT):
            if j not in waited:
                scatter(j).wait()

    slow = pack_slow(src).reshape(B, C, S, H, W)
    return (slow, frames)


# speedup vs baseline: 1.0061x; 1.0061x over previous
import functools

import jax
from jax import lax
from jax.experimental import pallas as pl
from jax.experimental.pallas import tpu as pltpu
from jax.experimental.pallas import tpu_sc as plsc


def kernel(frames):
    B, C, T, H, W = frames.shape
    S = T // 4
    BC = B * C
    ROWS = BC * S
    NW = 32
    RPW = ROWS // NW

    src = frames.reshape(BC, T, H, W)
    mesh = plsc.VectorSubcoreMesh(core_axis_name="c", subcore_axis_name="s")

    @functools.partial(
        pl.kernel,
        out_type=jax.ShapeDtypeStruct((BC, S, H, W), frames.dtype),
        mesh=mesh,
        scratch_types=[
            pltpu.VMEM((2, H, W), frames.dtype),
            pltpu.SemaphoreType.DMA((2,)),
            pltpu.SemaphoreType.DMA((2,)),
        ],
    )
    def pack_slow(src_hbm, out_hbm, buf, sin, sout):
        wid = lax.axis_index("s") * 2 + lax.axis_index("c")
        base = wid * RPW

        def gather(i):
            r = base + i
            tp = r % S
            return pltpu.make_async_copy(
                src_hbm.at[r // S, (tp * 567) >> 7],
                buf.at[i % 2], sin.at[i % 2])

        def scatter(i):
            r = base + i
            return pltpu.make_async_copy(
                buf.at[i % 2], out_hbm.at[r // S, r % S], sout.at[i % 2])

        gather(0).start()
        for i in range(RPW):
            if i + 1 < RPW:
                if i >= 1:
                    scatter(i - 1).wait()
                gather(i + 1).start()
            gather(i).wait()
            scatter(i).start()
        scatter(RPW - 2).wait()
        scatter(RPW - 1).wait()

    slow = pack_slow(src).reshape(B, C, S, H, W)
    return (slow, frames)
